# Initial kernel scaffold; baseline (speedup 1.0000x reference)
import jax
import jax.numpy as jnp
from jax.experimental import pallas as pl


def kernel(rel_pos_emb, rel_pos_bias, seq_length):
    raise NotImplementedError("write your pallas kernel here")


# speedup vs baseline: 26.6264x; 26.6264x over previous
import functools

import jax
import jax.numpy as jnp
from jax import lax
from jax.experimental import pallas as pl
from jax.experimental.pallas import tpu as pltpu
from jax.experimental.pallas import tpu_sc as plsc

_MAX_LENGTH = 2048
_SEQ = 2048
_NROWS = 2 * _MAX_LENGTH - 1
_D = 32
_NC = 2
_NS = 16
_NW = _NC * _NS
_ROWS_PER_W = _SEQ // _NW
_DEPTH = 4


def _sc_expand(rev, b8):
    mesh = plsc.VectorSubcoreMesh(core_axis_name="c", subcore_axis_name="s")

    @functools.partial(
        pl.kernel,
        mesh=mesh,
        out_type=[
            jax.ShapeDtypeStruct((_SEQ, _SEQ, _D), jnp.float32),
            jax.ShapeDtypeStruct((_SEQ, _SEQ), jnp.float32),
        ],
        scratch_types=[
            pltpu.VMEM_SHARED((4096, _D), jnp.float32),
            pltpu.VMEM_SHARED((8, 4096), jnp.float32),
            pltpu.SemaphoreType.DMA,
            pltpu.SemaphoreType.DMA,
        ],
        compiler_params=pltpu.CompilerParams(use_tc_tiling_on_sc=False),
    )
    def k(rev_hbm, b8_hbm, out_emb, out_bias, sh_rev, sh_b8, sem_e, sem_b):
        c = lax.axis_index("c")
        s = lax.axis_index("s")
        wid = s * _NC + c

        @pl.when(s == 0)
        def _fill():
            pltpu.sync_copy(rev_hbm, sh_rev)
            pltpu.sync_copy(b8_hbm, sh_b8)

        plsc.subcore_barrier()

        base = wid * _ROWS_PER_W

        def _emb_cp(i):
            off = (_SEQ - 1) - i
            return pltpu.make_async_copy(
                sh_rev.at[pl.ds(off, _SEQ)], out_emb.at[i], sem_e)

        def _bias_cp(i):
            off = (_SEQ - 1) - i
            p = lax.rem(off, 8)
            m = pl.multiple_of(off - p, 8)
            return pltpu.make_async_copy(
                sh_b8.at[p, pl.ds(m, _SEQ)], out_bias.at[i], sem_b)

        def body(i, _):
            _emb_cp(i).start()
            _bias_cp(i).start()

            @pl.when(i >= base + _DEPTH)
            def _drain_one():
                _emb_cp(i - _DEPTH).wait()
                _bias_cp(i - _DEPTH).wait()

            return 0

        lax.fori_loop(base, base + _ROWS_PER_W, body, 0)

        def tail(i, _):
            _emb_cp(i).wait()
            _bias_cp(i).wait()
            return 0

        lax.fori_loop(base + _ROWS_PER_W - _DEPTH, base + _ROWS_PER_W, tail, 0)

    return k(rev, b8)


def kernel(rel_pos_emb, rel_pos_bias, seq_length):
    shift = jnp.asarray(seq_length, jnp.int32) - _SEQ
    emb = jnp.roll(rel_pos_emb, -shift, axis=0)
    bias = jnp.roll(rel_pos_bias, -shift, axis=0)

    rev = jnp.flip(emb, axis=0)
    rev = jnp.concatenate([rev, jnp.zeros((1, _D), rev.dtype)], axis=0)

    brev = jnp.flip(bias)
    brevp = jnp.concatenate([brev, jnp.zeros((9,), brev.dtype)])
    b8 = jnp.stack(
        [lax.dynamic_slice_in_dim(brevp, p, 4096) for p in range(8)])

    out_emb, out_bias = _sc_expand(rev, b8)
    return (out_emb, out_bias)


# speedup vs baseline: 66.0056x; 2.4790x over previous
import functools

import jax
import jax.numpy as jnp
from jax import lax
from jax.experimental import pallas as pl
from jax.experimental.pallas import tpu as pltpu
from jax.experimental.pallas import tpu_sc as plsc

_MAX_LENGTH = 2048
_SEQ = 2048
_NROWS = 2 * _MAX_LENGTH - 1
_D = 32
_NC = 2
_NS = 16
_NW = _NC * _NS
_ROWS_PER_W = _SEQ // _NW
_DEPTH = 4


def _sc_expand(revT8, b8):
    mesh = plsc.VectorSubcoreMesh(core_axis_name="c", subcore_axis_name="s")

    @functools.partial(
        pl.kernel,
        mesh=mesh,
        out_type=[
            jax.ShapeDtypeStruct((_SEQ, _D, _SEQ), jnp.float32),
            jax.ShapeDtypeStruct((_SEQ, _SEQ), jnp.float32),
        ],
        scratch_types=[
            pltpu.VMEM_SHARED((8, _D, 4096), jnp.float32),
            pltpu.VMEM_SHARED((8, 4096), jnp.float32),
            pltpu.SemaphoreType.DMA,
            pltpu.SemaphoreType.DMA,
        ],
        compiler_params=pltpu.CompilerParams(use_tc_tiling_on_sc=False),
    )
    def k(revT8_hbm, b8_hbm, out_emb, out_bias, sh_revT8, sh_b8, sem_e, sem_b):
        c = lax.axis_index("c")
        s = lax.axis_index("s")
        wid = s * _NC + c

        @pl.when(s == 0)
        def _fill():
            pltpu.sync_copy(revT8_hbm, sh_revT8)
            pltpu.sync_copy(b8_hbm, sh_b8)

        plsc.subcore_barrier()

        base = wid * _ROWS_PER_W

        def _emb_cp(i):
            off = (_SEQ - 1) - i
            p = lax.rem(off, 8)
            m = pl.multiple_of(off - p, 8)
            return pltpu.make_async_copy(
                sh_revT8.at[p, :, pl.ds(m, _SEQ)], out_emb.at[i], sem_e)

        def _bias_cp(i):
            off = (_SEQ - 1) - i
            p = lax.rem(off, 8)
            m = pl.multiple_of(off - p, 8)
            return pltpu.make_async_copy(
                sh_b8.at[p, pl.ds(m, _SEQ)], out_bias.at[i], sem_b)

        def body(i, _):
            _emb_cp(i).start()
            _bias_cp(i).start()

            @pl.when(i >= base + _DEPTH)
            def _drain_one():
                _emb_cp(i - _DEPTH).wait()
                _bias_cp(i - _DEPTH).wait()

            return 0

        lax.fori_loop(base, base + _ROWS_PER_W, body, 0)

        def tail(i, _):
            _emb_cp(i).wait()
            _bias_cp(i).wait()
            return 0

        lax.fori_loop(base + _ROWS_PER_W - _DEPTH, base + _ROWS_PER_W, tail, 0)

    return k(revT8, b8)


def kernel(rel_pos_emb, rel_pos_bias, seq_length):
    shift = jnp.asarray(seq_length, jnp.int32) - _SEQ
    emb = jnp.roll(rel_pos_emb, -shift, axis=0)
    bias = jnp.roll(rel_pos_bias, -shift, axis=0)

    revT = jnp.flip(emb, axis=0).T
    revTpad = jnp.pad(revT, ((0, 0), (0, 9)))
    revT8 = jnp.stack(
        [lax.dynamic_slice_in_dim(revTpad, p, 4096, axis=1) for p in range(8)])

    brev = jnp.flip(bias)
    brevp = jnp.pad(brev, (0, 9))
    b8 = jnp.stack(
        [lax.dynamic_slice_in_dim(brevp, p, 4096) for p in range(8)])

    out_p, out_bias = _sc_expand(revT8, b8)
    return (jnp.transpose(out_p, (0, 2, 1)), out_bias)


# speedup vs baseline: 174.5475x; 2.6444x over previous
import functools

import jax
import jax.numpy as jnp
from jax import lax
from jax.experimental import pallas as pl
from jax.experimental.pallas import tpu as pltpu
from jax.experimental.pallas import tpu_sc as plsc

_MAX_LENGTH = 2048
_SEQ = 2048
_NROWS = 2 * _MAX_LENGTH - 1
_D = 32
_NC = 2
_NS = 16
_NW = _NC * _NS
_ROWS_PER_W = _SEQ // _NW
_DEPTH = 4


def _sc_expand(revT8, b8):
    mesh = plsc.VectorSubcoreMesh(core_axis_name="c", subcore_axis_name="s")

    @functools.partial(
        pl.kernel,
        mesh=mesh,
        out_type=[
            jax.ShapeDtypeStruct((_SEQ, 4, 16, 8, 128), jnp.float32),
            jax.ShapeDtypeStruct((_SEQ, _SEQ), jnp.float32),
        ],
        scratch_types=[
            pltpu.VMEM_SHARED((8, _D, 4096), jnp.float32),
            pltpu.VMEM_SHARED((8, 4096), jnp.float32),
            pltpu.SemaphoreType.DMA,
            pltpu.SemaphoreType.DMA,
        ],
        compiler_params=pltpu.CompilerParams(use_tc_tiling_on_sc=False),
    )
    def k(revT8_hbm, b8_hbm, out_emb, out_bias, sh_revT8, sh_b8, sem_e, sem_b):
        c = lax.axis_index("c")
        s = lax.axis_index("s")
        wid = s * _NC + c

        @pl.when(s == 0)
        def _fill():
            pltpu.sync_copy(revT8_hbm, sh_revT8)
            pltpu.sync_copy(b8_hbm, sh_b8)

        plsc.subcore_barrier()

        base = wid * _ROWS_PER_W

        def _start_plane(i):
            off = (_SEQ - 1) - i
            p = lax.rem(off, 8)
            m = off - p
            for td in range(4):
                for tj in range(16):
                    src = sh_revT8.at[
                        p, pl.ds(8 * td, 8),
                        pl.ds(pl.multiple_of(m + 128 * tj, 8), 128)]
                    pltpu.make_async_copy(src, out_emb.at[i, td, tj],
                                          sem_e).start()

        def _wait_plane():
            def w(j, _):
                pltpu.make_async_copy(
                    sh_revT8.at[0, pl.ds(0, 8), pl.ds(0, 128)],
                    out_emb.at[0, 0, 0], sem_e).wait()
                return 0
            lax.fori_loop(0, 64, w, 0, unroll=8)

        def _bias_cp(i):
            off = (_SEQ - 1) - i
            p = lax.rem(off, 8)
            m = pl.multiple_of(off - p, 8)
            return pltpu.make_async_copy(
                sh_b8.at[p, pl.ds(m, _SEQ)], out_bias.at[i], sem_b)

        def body(i, _):
            _start_plane(i)
            _bias_cp(i).start()

            @pl.when(i > base)
            def _drain_prev():
                _wait_plane()

            @pl.when(i >= base + _DEPTH)
            def _drain_bias():
                _bias_cp(i - _DEPTH).wait()

            return 0

        lax.fori_loop(base, base + _ROWS_PER_W, body, 0)
        _wait_plane()

        def tail(i, _):
            _bias_cp(i).wait()
            return 0

        lax.fori_loop(base + _ROWS_PER_W - _DEPTH, base + _ROWS_PER_W,
                      tail, 0)

    return k(revT8, b8)


def kernel(rel_pos_emb, rel_pos_bias, seq_length):
    shift = jnp.asarray(seq_length, jnp.int32) - _SEQ
    emb = jnp.roll(rel_pos_emb, -shift, axis=0)
    bias = jnp.roll(rel_pos_bias, -shift, axis=0)

    revT = jnp.flip(emb, axis=0).T
    revTpad = jnp.pad(revT, ((0, 0), (0, 9)))
    revT8 = jnp.stack(
        [lax.dynamic_slice_in_dim(revTpad, p, 4096, axis=1) for p in range(8)])

    brev = jnp.flip(bias)
    brevp = jnp.pad(brev, (0, 9))
    b8 = jnp.stack(
        [lax.dynamic_slice_in_dim(brevp, p, 4096) for p in range(8)])

    out5, out_bias = _sc_expand(revT8, b8)
    x = jnp.transpose(out5, (0, 1, 3, 2, 4)).reshape(_SEQ, _D, _SEQ)
    return (jnp.transpose(x, (0, 2, 1)), out_bias)
